# Initial kernel scaffold; baseline (speedup 1.0000x reference)
import jax
import jax.numpy as jnp
from jax.experimental import pallas as pl


def kernel(users, movies, genres, tags, lang, budget, popularity, runtime, vote_average, vote_count, revenue, W_user, W_movie, W_tag, W1, b1, W2, b2):
    raise NotImplementedError("write your pallas kernel here")


# speedup vs baseline: 2.0744x; 2.0744x over previous
import functools

import jax
import jax.numpy as jnp
from jax import lax
from jax.experimental import pallas as pl
from jax.experimental.pallas import tpu as pltpu
from jax.experimental.pallas import tpu_sc as plsc

NC = 2
NS = 16
NW = NC * NS
CH = 128


def _sc_gather_call(midx2d, tidx2d, seg2d, zeros, w_movie_p, w_tag_p, b, hist):
    pw = b // NW
    hp = pw // 2
    mch = pw // CH
    tch = pw * hist // CH
    half = tch // 2
    acc_rows = NS * hp

    mesh = plsc.VectorSubcoreMesh(
        core_axis_name="c", subcore_axis_name="s", num_cores=NC, num_subcores=NS
    )

    @functools.partial(
        pl.kernel,
        out_type=(
            jax.ShapeDtypeStruct((b, 128), jnp.float32),
            jax.ShapeDtypeStruct((b, 128), jnp.float32),
        ),
        mesh=mesh,
        scratch_types=(
            pltpu.VMEM((tch, CH), jnp.int32),
            pltpu.VMEM((tch, CH), jnp.int32),
            pltpu.VMEM((mch, CH), jnp.int32),
            pltpu.VMEM((2 * CH, 128), jnp.float32),
            pltpu.VMEM_SHARED((acc_rows, 128), jnp.float32),
            pltpu.SemaphoreType.DMA,
            pltpu.SemaphoreType.DMA,
        ),
    )
    def sc_kernel(midx_h, tidx_h, seg_h, zeros_h, wm_h, wt_h,
                  mout_h, tout_h,
                  tidx_v, seg_v, midx_v, ring_v, acc_sh, sem0, sem1):
        c = lax.axis_index("c")
        s = lax.axis_index("s")
        wid = c * NS + s
        base = wid * pw
        slab = s * hp

        slot0 = ring_v.at[pl.ds(0, CH)]
        slot1 = ring_v.at[pl.ds(CH, CH)]

        pltpu.sync_copy(midx_h.at[pl.ds(wid * mch, mch)], midx_v)
        pltpu.async_copy(wm_h.at[midx_v.at[0]], slot0, sem0)
        pltpu.async_copy(wm_h.at[midx_v.at[1]], slot1, sem1)
        for j in range(mch):
            slot = slot0 if j % 2 == 0 else slot1
            sem = sem0 if j % 2 == 0 else sem1
            pltpu.make_async_copy(wm_h.at[midx_v.at[0]], slot, sem).wait()
            pltpu.sync_copy(slot, mout_h.at[pl.ds(base + j * CH, CH)])
            if j + 2 < mch:
                pltpu.async_copy(wm_h.at[midx_v.at[j + 2]], slot, sem)

        pltpu.sync_copy(tidx_h.at[pl.ds(wid * tch, tch)], tidx_v)
        pltpu.sync_copy(seg_h.at[pl.ds(wid * tch, tch)], seg_v)

        for p in range(2):
            pltpu.sync_copy(zeros_h, acc_sh.at[pl.ds(slab, hp)])
            pb = p * half
            pltpu.async_copy(wt_h.at[tidx_v.at[pb]], slot0, sem0)
            pltpu.async_copy(wt_h.at[tidx_v.at[pb + 1]], slot1, sem1)

            def chunk_step(i, carry):
                k = pb + 2 * i
                pltpu.make_async_copy(wt_h.at[tidx_v.at[0]], slot0, sem0).wait()
                pltpu.sync_copy(slot0, acc_sh.at[seg_v.at[k]], add=True)
                pltpu.async_copy(wt_h.at[tidx_v.at[k + 2]], slot0, sem0)
                pltpu.make_async_copy(wt_h.at[tidx_v.at[0]], slot1, sem1).wait()
                pltpu.sync_copy(slot1, acc_sh.at[seg_v.at[k + 1]], add=True)
                pltpu.async_copy(wt_h.at[tidx_v.at[k + 3]], slot1, sem1)
                return carry

            lax.fori_loop(0, half // 2 - 1, chunk_step, 0)
            pltpu.make_async_copy(wt_h.at[tidx_v.at[0]], slot0, sem0).wait()
            pltpu.sync_copy(slot0, acc_sh.at[seg_v.at[pb + half - 2]], add=True)
            pltpu.make_async_copy(wt_h.at[tidx_v.at[0]], slot1, sem1).wait()
            pltpu.sync_copy(slot1, acc_sh.at[seg_v.at[pb + half - 1]], add=True)

            pltpu.sync_copy(acc_sh.at[pl.ds(slab, hp)],
                            tout_h.at[pl.ds(base + p * hp, hp)])

    return sc_kernel(midx2d, tidx2d, seg2d, zeros, w_movie_p, w_tag_p)


def _mlp_call(u, m, g, t, sc_, w1u, w1m, w1g, w1t, w1s, b1r, w2r, b2r, inv_hist):
    b = u.shape[0]
    rb = 2048
    grid = (b // rb,)

    def body(u_r, m_r, g_r, t_r, s_r, w1u_r, w1m_r, w1g_r, w1t_r, w1s_r,
             b1_r, w2_r, b2_r, o_r):
        h = jnp.dot(u_r[...], w1u_r[...], preferred_element_type=jnp.float32)
        h = h + jnp.dot(m_r[...], w1m_r[...], preferred_element_type=jnp.float32)
        h = h + jnp.dot(g_r[...], w1g_r[...], preferred_element_type=jnp.float32)
        h = h + jnp.dot(t_r[...] * inv_hist, w1t_r[...],
                        preferred_element_type=jnp.float32)
        h = h + jnp.dot(s_r[...], w1s_r[...], preferred_element_type=jnp.float32)
        h = jnp.maximum(h + b1_r[...], 0.0)
        logit = jnp.sum(h * w2_r[...], axis=1) + b2_r[0, 0]
        o_r[...] = jax.nn.sigmoid(logit) * 4.5 + 0.5

    def rowspec(d):
        return pl.BlockSpec((rb, d), lambda i: (i, 0))

    def fullspec(r, d):
        return pl.BlockSpec((r, d), lambda i: (0, 0))

    return pl.pallas_call(
        body,
        grid=grid,
        in_specs=[
            rowspec(u.shape[1]), rowspec(m.shape[1]), rowspec(g.shape[1]),
            rowspec(t.shape[1]), rowspec(sc_.shape[1]),
            fullspec(*w1u.shape), fullspec(*w1m.shape), fullspec(*w1g.shape),
            fullspec(*w1t.shape), fullspec(*w1s.shape),
            fullspec(*b1r.shape), fullspec(*w2r.shape), fullspec(*b2r.shape),
        ],
        out_specs=pl.BlockSpec((rb,), lambda i: (i,)),
        out_shape=jax.ShapeDtypeStruct((b,), jnp.float32),
    )(u, m, g, t, sc_, w1u, w1m, w1g, w1t, w1s, b1r, w2r, b2r)


def kernel(users, movies, genres, tags, lang, budget, popularity, runtime,
           vote_average, vote_count, revenue, W_user, W_movie, W_tag, W1, b1, W2, b2):
    b = users.shape[0]
    hist = tags.shape[1]
    d_um = W_user.shape[1]
    d_tag = W_tag.shape[1]
    ng = genres.shape[1]
    sc_rows = b // NC
    pw = b // NW

    pad_m = [(0, 0, 0), (0, 128 - d_um, 0)]
    pad_t = [(0, 0, 0), (0, 128 - d_tag, 0)]
    w_movie_p = lax.pad(W_movie, jnp.float32(0), pad_m)
    w_tag_p = lax.pad(W_tag, jnp.float32(0), pad_t)

    midx2d = movies.astype(jnp.int32).reshape(-1, CH)
    tidx2d = tags.astype(jnp.int32).reshape(-1, CH)
    s_arr = jnp.arange(b, dtype=jnp.int32)
    seg = ((s_arr % (b // NC)) // (b // NW)) * (b // NW // 2) + (s_arr % (b // NW // 2))
    seg2d = jnp.repeat(seg, hist).reshape(-1, CH)
    zeros = jnp.zeros((b // NW // 2, 128), jnp.float32)

    movie_emb, tag_sum = _sc_gather_call(
        midx2d, tidx2d, seg2d, zeros, w_movie_p, w_tag_p, b, hist)

    user_emb = jnp.take(W_user, users.astype(jnp.int32), axis=0)

    scal = jnp.stack(
        [lang.astype(jnp.float32), budget, popularity, runtime,
         vote_average, vote_count, revenue], axis=1)

    w1u = W1[:, :d_um].T
    w1m = jnp.zeros((128, 128), jnp.float32).at[:d_um].set(W1[:, d_um:2 * d_um].T)
    w1g = W1[:, 2 * d_um:2 * d_um + ng].T
    w1t = jnp.zeros((128, 128), jnp.float32).at[:d_tag].set(
        W1[:, 2 * d_um + ng:2 * d_um + ng + d_tag].T)
    w1s = W1[:, 2 * d_um + ng + d_tag:].T
    b1r = b1.reshape(1, -1)
    b2r = b2.reshape(1, 1)

    return _mlp_call(user_emb, movie_emb, genres, tag_sum, scal,
                     w1u, w1m, w1g, w1t, w1s, b1r, W2, b2r, 1.0 / hist)


# speedup vs baseline: 2.3175x; 1.1172x over previous
import functools

import jax
import jax.numpy as jnp
from jax import lax
from jax.experimental import pallas as pl
from jax.experimental.pallas import tpu as pltpu
from jax.experimental.pallas import tpu_sc as plsc

NC = 2
NS = 16
NW = NC * NS
CH = 128


def _sc_gather_call(midx2d, tidx2d, seg2d, zeros, w_movie_p, w_tag_p, b, hist):
    pw = b // NW
    hp = pw // 2
    mch = pw // CH
    tch = pw * hist // CH
    half = tch // 2
    acc_rows = NS * hp

    mesh = plsc.VectorSubcoreMesh(
        core_axis_name="c", subcore_axis_name="s", num_cores=NC, num_subcores=NS
    )

    @functools.partial(
        pl.kernel,
        out_type=(
            jax.ShapeDtypeStruct((b, 128), jnp.float32),
            jax.ShapeDtypeStruct((b, 128), jnp.float32),
        ),
        mesh=mesh,
        scratch_types=(
            pltpu.VMEM((tch, CH), jnp.int32),
            pltpu.VMEM((tch, CH), jnp.int32),
            pltpu.VMEM((mch, CH), jnp.int32),
            pltpu.VMEM((2 * CH, 128), jnp.float32),
            pltpu.VMEM_SHARED((acc_rows, 128), jnp.float32),
            pltpu.SemaphoreType.DMA,
            pltpu.SemaphoreType.DMA,
        ),
    )
    def sc_kernel(midx_h, tidx_h, seg_h, zeros_h, wm_h, wt_h,
                  mout_h, tout_h,
                  tidx_v, seg_v, midx_v, ring_v, acc_sh, sem0, sem1):
        c = lax.axis_index("c")
        s = lax.axis_index("s")
        wid = c * NS + s
        base = wid * pw
        slab = s * hp

        slot0 = ring_v.at[pl.ds(0, CH)]
        slot1 = ring_v.at[pl.ds(CH, CH)]

        pltpu.sync_copy(midx_h.at[pl.ds(wid * mch, mch)], midx_v)
        pltpu.async_copy(wm_h.at[midx_v.at[0]], slot0, sem0)
        pltpu.async_copy(wm_h.at[midx_v.at[1]], slot1, sem1)
        for j in range(mch):
            slot = slot0 if j % 2 == 0 else slot1
            sem = sem0 if j % 2 == 0 else sem1
            pltpu.make_async_copy(wm_h.at[midx_v.at[0]], slot, sem).wait()
            pltpu.sync_copy(slot, mout_h.at[pl.ds(base + j * CH, CH)])
            if j + 2 < mch:
                pltpu.async_copy(wm_h.at[midx_v.at[j + 2]], slot, sem)

        pltpu.sync_copy(tidx_h.at[pl.ds(wid * tch, tch)], tidx_v)
        pltpu.sync_copy(seg_h.at[pl.ds(wid * tch, tch)], seg_v)

        for p in range(2):
            pltpu.sync_copy(zeros_h, acc_sh.at[pl.ds(slab, hp)])
            pb = p * half
            pltpu.async_copy(wt_h.at[tidx_v.at[pb]], slot0, sem0)
            pltpu.async_copy(wt_h.at[tidx_v.at[pb + 1]], slot1, sem1)

            def chunk_step(i, carry):
                k = pb + 2 * i
                pltpu.make_async_copy(wt_h.at[tidx_v.at[0]], slot0, sem0).wait()
                pltpu.sync_copy(slot0, acc_sh.at[seg_v.at[k]], add=True)
                pltpu.async_copy(wt_h.at[tidx_v.at[k + 2]], slot0, sem0)
                pltpu.make_async_copy(wt_h.at[tidx_v.at[0]], slot1, sem1).wait()
                pltpu.sync_copy(slot1, acc_sh.at[seg_v.at[k + 1]], add=True)
                pltpu.async_copy(wt_h.at[tidx_v.at[k + 3]], slot1, sem1)
                return carry

            lax.fori_loop(0, half // 2 - 1, chunk_step, 0)
            pltpu.make_async_copy(wt_h.at[tidx_v.at[0]], slot0, sem0).wait()
            pltpu.sync_copy(slot0, acc_sh.at[seg_v.at[pb + half - 2]], add=True)
            pltpu.make_async_copy(wt_h.at[tidx_v.at[0]], slot1, sem1).wait()
            pltpu.sync_copy(slot1, acc_sh.at[seg_v.at[pb + half - 1]], add=True)

            pltpu.sync_copy(acc_sh.at[pl.ds(slab, hp)],
                            tout_h.at[pl.ds(base + p * hp, hp)])

    return sc_kernel(midx2d, tidx2d, seg2d, zeros, w_movie_p, w_tag_p)


def _pad_call(w, rows_per_block):
    v, d = w.shape
    grid = (v // rows_per_block,)

    def body(w_r, o_r):
        o_r[...] = jnp.concatenate(
            [w_r[...], jnp.zeros((rows_per_block, 128 - d), jnp.float32)], axis=1)

    return pl.pallas_call(
        body,
        grid=grid,
        in_specs=[pl.BlockSpec((rows_per_block, d), lambda i: (i, 0))],
        out_specs=pl.BlockSpec((rows_per_block, 128), lambda i: (i, 0)),
        out_shape=jax.ShapeDtypeStruct((v, 128), jnp.float32),
    )(w)


def _mlp_call(u, m, g, t, sc_, w1u, w1m, w1g, w1t, w1s, b1r, w2r, b2r, inv_hist):
    b = u.shape[0]
    rb = 2048
    grid = (b // rb,)

    def body(u_r, m_r, g_r, t_r, s_r, w1u_r, w1m_r, w1g_r, w1t_r, w1s_r,
             b1_r, w2_r, b2_r, o_r):
        h = jnp.dot(u_r[...], w1u_r[...], preferred_element_type=jnp.float32)
        h = h + jnp.dot(m_r[...], w1m_r[...], preferred_element_type=jnp.float32)
        h = h + jnp.dot(g_r[...], w1g_r[...], preferred_element_type=jnp.float32)
        h = h + jnp.dot(t_r[...] * inv_hist, w1t_r[...],
                        preferred_element_type=jnp.float32)
        h = h + jnp.dot(s_r[...], w1s_r[...], preferred_element_type=jnp.float32)
        h = jnp.maximum(h + b1_r[...], 0.0)
        logit = jnp.sum(h * w2_r[...], axis=1) + b2_r[0, 0]
        o_r[...] = jax.nn.sigmoid(logit) * 4.5 + 0.5

    def rowspec(d):
        return pl.BlockSpec((rb, d), lambda i: (i, 0))

    def fullspec(r, d):
        return pl.BlockSpec((r, d), lambda i: (0, 0))

    return pl.pallas_call(
        body,
        grid=grid,
        in_specs=[
            rowspec(u.shape[1]), rowspec(m.shape[1]), rowspec(g.shape[1]),
            rowspec(t.shape[1]), rowspec(sc_.shape[1]),
            fullspec(*w1u.shape), fullspec(*w1m.shape), fullspec(*w1g.shape),
            fullspec(*w1t.shape), fullspec(*w1s.shape),
            fullspec(*b1r.shape), fullspec(*w2r.shape), fullspec(*b2r.shape),
        ],
        out_specs=pl.BlockSpec((rb,), lambda i: (i,)),
        out_shape=jax.ShapeDtypeStruct((b,), jnp.float32),
    )(u, m, g, t, sc_, w1u, w1m, w1g, w1t, w1s, b1r, w2r, b2r)


def kernel(users, movies, genres, tags, lang, budget, popularity, runtime,
           vote_average, vote_count, revenue, W_user, W_movie, W_tag, W1, b1, W2, b2):
    b = users.shape[0]
    hist = tags.shape[1]
    d_um = W_user.shape[1]
    d_tag = W_tag.shape[1]
    ng = genres.shape[1]
    sc_rows = b // NC
    pw = b // NW

    w_movie_p = _pad_call(W_movie, 2000)
    w_tag_p = _pad_call(W_tag, 2000)

    midx2d = movies.astype(jnp.int32).reshape(-1, CH)
    tidx2d = tags.astype(jnp.int32).reshape(-1, CH)
    l_arr = lax.broadcasted_iota(jnp.int32, (b * hist // CH, CH), 0) * CH + \
        lax.broadcasted_iota(jnp.int32, (b * hist // CH, CH), 1)
    s_of_l = l_arr // hist
    seg2d = ((s_of_l % (b // NC)) // (b // NW)) * (b // NW // 2) + \
        (s_of_l % (b // NW // 2))
    zeros = jnp.zeros((b // NW // 2, 128), jnp.float32)

    movie_emb, tag_sum = _sc_gather_call(
        midx2d, tidx2d, seg2d, zeros, w_movie_p, w_tag_p, b, hist)

    user_emb = jnp.take(W_user, users.astype(jnp.int32), axis=0)

    scal = jnp.stack(
        [lang.astype(jnp.float32), budget, popularity, runtime,
         vote_average, vote_count, revenue], axis=1)

    w1u = W1[:, :d_um].T
    w1m = jnp.zeros((128, 128), jnp.float32).at[:d_um].set(W1[:, d_um:2 * d_um].T)
    w1g = W1[:, 2 * d_um:2 * d_um + ng].T
    w1t = jnp.zeros((128, 128), jnp.float32).at[:d_tag].set(
        W1[:, 2 * d_um + ng:2 * d_um + ng + d_tag].T)
    w1s = W1[:, 2 * d_um + ng + d_tag:].T
    b1r = b1.reshape(1, -1)
    b2r = b2.reshape(1, 1)

    return _mlp_call(user_emb, movie_emb, genres, tag_sum, scal,
                     w1u, w1m, w1g, w1t, w1s, b1r, W2, b2r, 1.0 / hist)


# speedup vs baseline: 6.5097x; 2.8089x over previous
import functools

import jax
import jax.numpy as jnp
from jax import lax
from jax.experimental import pallas as pl
from jax.experimental.pallas import tpu as pltpu
from jax.experimental.pallas import tpu_sc as plsc

NC = 2
NS = 16
NW = NC * NS
CH = 128


def _sc_gather_call(midx2d, tidx2d, seg2d, zeros, w_movie_p, w_tag_p, b, hist):
    pw = b // NW
    hp = pw // 2
    mch = pw // CH
    tch = pw * hist // CH
    half = tch // 2
    acc_rows = NS * hp

    mesh = plsc.VectorSubcoreMesh(
        core_axis_name="c", subcore_axis_name="s", num_cores=NC, num_subcores=NS
    )

    @functools.partial(
        pl.kernel,
        out_type=(
            jax.ShapeDtypeStruct((b, 128), jnp.float32),
            jax.ShapeDtypeStruct((b, 128), jnp.float32),
        ),
        mesh=mesh,
        scratch_types=(
            pltpu.VMEM((tch, CH), jnp.int32),
            pltpu.VMEM((tch, CH), jnp.int32),
            pltpu.VMEM((mch, CH), jnp.int32),
            pltpu.VMEM((2 * CH, 128), jnp.float32),
            pltpu.VMEM_SHARED((acc_rows, 128), jnp.float32),
            pltpu.SemaphoreType.DMA,
            pltpu.SemaphoreType.DMA,
        ),
    )
    def sc_kernel(midx_h, tidx_h, seg_h, zeros_h, wm_h, wt_h,
                  mout_h, tout_h,
                  tidx_v, seg_v, midx_v, ring_v, acc_sh, sem0, sem1):
        c = lax.axis_index("c")
        s = lax.axis_index("s")
        wid = c * NS + s
        base = wid * pw
        slab = s * hp

        slot0 = ring_v.at[pl.ds(0, CH)]
        slot1 = ring_v.at[pl.ds(CH, CH)]

        pltpu.sync_copy(midx_h.at[pl.ds(wid * mch, mch)], midx_v)
        pltpu.async_copy(wm_h.at[midx_v.at[0]], slot0, sem0)
        pltpu.async_copy(wm_h.at[midx_v.at[1]], slot1, sem1)
        for j in range(mch):
            slot = slot0 if j % 2 == 0 else slot1
            sem = sem0 if j % 2 == 0 else sem1
            pltpu.make_async_copy(wm_h.at[midx_v.at[0]], slot, sem).wait()
            pltpu.sync_copy(slot, mout_h.at[pl.ds(base + j * CH, CH)])
            if j + 2 < mch:
                pltpu.async_copy(wm_h.at[midx_v.at[j + 2]], slot, sem)

        pltpu.sync_copy(tidx_h.at[pl.ds(wid * tch, tch)], tidx_v)
        pltpu.sync_copy(seg_h.at[pl.ds(wid * tch, tch)], seg_v)

        for p in range(2):
            pltpu.sync_copy(zeros_h, acc_sh.at[pl.ds(slab, hp)])
            pb = p * half
            pltpu.async_copy(wt_h.at[tidx_v.at[pb]], slot0, sem0)
            pltpu.async_copy(wt_h.at[tidx_v.at[pb + 1]], slot1, sem1)

            def chunk_step(i, carry):
                k = pb + 2 * i
                pltpu.make_async_copy(wt_h.at[tidx_v.at[0]], slot0, sem0).wait()
                pltpu.sync_copy(slot0, acc_sh.at[seg_v.at[k]], add=True)
                pltpu.async_copy(wt_h.at[tidx_v.at[k + 2]], slot0, sem0)
                pltpu.make_async_copy(wt_h.at[tidx_v.at[0]], slot1, sem1).wait()
                pltpu.sync_copy(slot1, acc_sh.at[seg_v.at[k + 1]], add=True)
                pltpu.async_copy(wt_h.at[tidx_v.at[k + 3]], slot1, sem1)
                return carry

            lax.fori_loop(0, half // 2 - 1, chunk_step, 0)
            pltpu.make_async_copy(wt_h.at[tidx_v.at[0]], slot0, sem0).wait()
            pltpu.sync_copy(slot0, acc_sh.at[seg_v.at[pb + half - 2]], add=True)
            pltpu.make_async_copy(wt_h.at[tidx_v.at[0]], slot1, sem1).wait()
            pltpu.sync_copy(slot1, acc_sh.at[seg_v.at[pb + half - 1]], add=True)

            pltpu.sync_copy(acc_sh.at[pl.ds(slab, hp)],
                            tout_h.at[pl.ds(base + p * hp, hp)])

    return sc_kernel(midx2d, tidx2d, seg2d, zeros, w_movie_p, w_tag_p)


def _pad_call(w, rows_per_block):
    v, d = w.shape
    grid = (v // rows_per_block,)

    def body(w_r, o_r):
        o_r[...] = jnp.concatenate(
            [w_r[...], jnp.zeros((rows_per_block, 128 - d), jnp.float32)], axis=1)

    return pl.pallas_call(
        body,
        grid=grid,
        in_specs=[pl.BlockSpec((rows_per_block, d), lambda i: (i, 0))],
        out_specs=pl.BlockSpec((rows_per_block, 128), lambda i: (i, 0)),
        out_shape=jax.ShapeDtypeStruct((v, 128), jnp.float32),
    )(w)


def _user_elem_gather(W_user, users):
    b = users.shape[0]
    d = W_user.shape[1]
    rows = jnp.broadcast_to(users.astype(jnp.int32)[:, None], (b, d))
    cols = jnp.broadcast_to(jnp.arange(d, dtype=jnp.int32)[None, :], (b, d))
    idx = jnp.stack([rows, cols], axis=-1).reshape(b * d, 2)
    dn = lax.GatherDimensionNumbers(
        offset_dims=(), collapsed_slice_dims=(0, 1), start_index_map=(0, 1))
    out = lax.gather(W_user, idx, dn, (1, 1),
                     mode=lax.GatherScatterMode.PROMISE_IN_BOUNDS)
    return out.reshape(b, d)


def _mlp_call(u, m, g, t, sc_, w1u, w1m, w1g, w1t, w1s, b1r, w2r, b2r, inv_hist):
    b = m.shape[0]
    rb = 2048
    grid = (b // rb,)

    def body(u_r, m_r, g_r, t_r, s_r, w1u_r, w1m_r, w1g_r, w1t_r, w1s_r,
             b1_r, w2_r, b2_r, o_r):
        h = jnp.dot(u_r[...], w1u_r[...], preferred_element_type=jnp.float32)
        h = h + jnp.dot(m_r[...], w1m_r[...], preferred_element_type=jnp.float32)
        h = h + jnp.dot(g_r[...], w1g_r[...], preferred_element_type=jnp.float32)
        h = h + jnp.dot(t_r[...] * inv_hist, w1t_r[...],
                        preferred_element_type=jnp.float32)
        h = h + jnp.dot(s_r[...], w1s_r[...], preferred_element_type=jnp.float32)
        h = jnp.maximum(h + b1_r[...], 0.0)
        logit = jnp.sum(h * w2_r[...], axis=1) + b2_r[0, 0]
        o_r[...] = jax.nn.sigmoid(logit) * 4.5 + 0.5

    def rowspec(d):
        return pl.BlockSpec((rb, d), lambda i: (i, 0))

    def fullspec(r, d):
        return pl.BlockSpec((r, d), lambda i: (0, 0))

    return pl.pallas_call(
        body,
        grid=grid,
        in_specs=[
            rowspec(u.shape[1]),
            rowspec(m.shape[1]), rowspec(g.shape[1]),
            rowspec(t.shape[1]), rowspec(sc_.shape[1]),
            fullspec(*w1u.shape), fullspec(*w1m.shape), fullspec(*w1g.shape),
            fullspec(*w1t.shape), fullspec(*w1s.shape),
            fullspec(*b1r.shape), fullspec(*w2r.shape), fullspec(*b2r.shape),
        ],
        out_specs=pl.BlockSpec((rb,), lambda i: (i,)),
        out_shape=jax.ShapeDtypeStruct((b,), jnp.float32),
    )(u, m, g, t, sc_, w1u, w1m, w1g, w1t, w1s, b1r, w2r, b2r)


def kernel(users, movies, genres, tags, lang, budget, popularity, runtime,
           vote_average, vote_count, revenue, W_user, W_movie, W_tag, W1, b1, W2, b2):
    b = users.shape[0]
    hist = tags.shape[1]
    d_um = W_user.shape[1]
    d_tag = W_tag.shape[1]
    ng = genres.shape[1]
    sc_rows = b // NC
    pw = b // NW

    w_movie_p = _pad_call(W_movie, 2000)
    w_tag_p = _pad_call(W_tag, 2000)

    midx2d = movies.astype(jnp.int32).reshape(-1, CH)
    tidx2d = tags.astype(jnp.int32).reshape(-1, CH)
    l_arr = lax.broadcasted_iota(jnp.int32, (b * hist // CH, CH), 0) * CH + \
        lax.broadcasted_iota(jnp.int32, (b * hist // CH, CH), 1)
    s_of_l = l_arr // hist
    seg2d = ((s_of_l % (b // NC)) // (b // NW)) * (b // NW // 2) + \
        (s_of_l % (b // NW // 2))
    zeros = jnp.zeros((b // NW // 2, 128), jnp.float32)

    movie_emb, tag_sum = _sc_gather_call(
        midx2d, tidx2d, seg2d, zeros, w_movie_p, w_tag_p, b, hist)

    user_emb = _user_elem_gather(W_user, users)

    scal = jnp.stack(
        [lang.astype(jnp.float32), budget, popularity, runtime,
         vote_average, vote_count, revenue], axis=1)

    w1u = W1[:, :d_um].T
    w1m = jnp.zeros((128, 128), jnp.float32).at[:d_um].set(W1[:, d_um:2 * d_um].T)
    w1g = W1[:, 2 * d_um:2 * d_um + ng].T
    w1t = jnp.zeros((128, 128), jnp.float32).at[:d_tag].set(
        W1[:, 2 * d_um + ng:2 * d_um + ng + d_tag].T)
    w1s = W1[:, 2 * d_um + ng + d_tag:].T
    b1r = b1.reshape(1, -1)
    b2r = b2.reshape(1, 1)

    return _mlp_call(user_emb, movie_emb, genres, tag_sum, scal,
                     w1u, w1m, w1g, w1t, w1s, b1r, W2, b2r, 1.0 / hist)


# speedup vs baseline: 7.3559x; 1.1300x over previous
import functools

import jax
import jax.numpy as jnp
from jax import lax
from jax.experimental import pallas as pl
from jax.experimental.pallas import tpu as pltpu
from jax.experimental.pallas import tpu_sc as plsc

NC = 2
NS = 16
NW = NC * NS
CH = 128


def _sc_gather_call(midx2d, tidx2d, seg2d, zeros, w_movie_p, w_tag_p, b, hist):
    pw = b // NW
    hp = pw // 2
    mch = pw // CH
    tch = pw * hist // CH
    half = tch // 2
    acc_rows = NS * hp

    mesh = plsc.VectorSubcoreMesh(
        core_axis_name="c", subcore_axis_name="s", num_cores=NC, num_subcores=NS
    )

    @functools.partial(
        pl.kernel,
        out_type=(
            jax.ShapeDtypeStruct((b, 128), jnp.float32),
            jax.ShapeDtypeStruct((b, 128), jnp.float32),
        ),
        mesh=mesh,
        scratch_types=(
            pltpu.VMEM((tch, CH), jnp.int32),
            pltpu.VMEM((tch, CH), jnp.int32),
            pltpu.VMEM((mch, CH), jnp.int32),
            pltpu.VMEM((2 * CH, 128), jnp.float32),
            pltpu.VMEM_SHARED((acc_rows, 128), jnp.float32),
            pltpu.SemaphoreType.DMA,
            pltpu.SemaphoreType.DMA,
        ),
    )
    def sc_kernel(midx_h, tidx_h, seg_h, zeros_h, wm_h, wt_h,
                  mout_h, tout_h,
                  tidx_v, seg_v, midx_v, ring_v, acc_sh, sem0, sem1):
        c = lax.axis_index("c")
        s = lax.axis_index("s")
        wid = c * NS + s
        base = wid * pw
        slab = s * hp

        slot0 = ring_v.at[pl.ds(0, CH)]
        slot1 = ring_v.at[pl.ds(CH, CH)]

        pltpu.sync_copy(midx_h.at[pl.ds(wid * mch, mch)], midx_v)
        pltpu.async_copy(wm_h.at[midx_v.at[0]], slot0, sem0)
        pltpu.async_copy(wm_h.at[midx_v.at[1]], slot1, sem1)
        for j in range(mch):
            slot = slot0 if j % 2 == 0 else slot1
            sem = sem0 if j % 2 == 0 else sem1
            pltpu.make_async_copy(wm_h.at[midx_v.at[0]], slot, sem).wait()
            pltpu.sync_copy(slot, mout_h.at[pl.ds(base + j * CH, CH)])
            if j + 2 < mch:
                pltpu.async_copy(wm_h.at[midx_v.at[j + 2]], slot, sem)

        pltpu.sync_copy(tidx_h.at[pl.ds(wid * tch, tch)], tidx_v)
        pltpu.sync_copy(seg_h.at[pl.ds(wid * tch, tch)], seg_v)

        for p in range(2):
            pltpu.sync_copy(zeros_h, acc_sh.at[pl.ds(slab, hp)])
            pb = p * half
            pltpu.async_copy(wt_h.at[tidx_v.at[pb]], slot0, sem0)
            pltpu.async_copy(wt_h.at[tidx_v.at[pb + 1]], slot1, sem1)

            def chunk_step(i, carry):
                k = pb + 2 * i
                pltpu.make_async_copy(wt_h.at[tidx_v.at[0]], slot0, sem0).wait()
                pltpu.sync_copy(slot0, acc_sh.at[seg_v.at[k]], add=True)
                pltpu.async_copy(wt_h.at[tidx_v.at[k + 2]], slot0, sem0)
                pltpu.make_async_copy(wt_h.at[tidx_v.at[0]], slot1, sem1).wait()
                pltpu.sync_copy(slot1, acc_sh.at[seg_v.at[k + 1]], add=True)
                pltpu.async_copy(wt_h.at[tidx_v.at[k + 3]], slot1, sem1)
                return carry

            lax.fori_loop(0, half // 2 - 1, chunk_step, 0)
            pltpu.make_async_copy(wt_h.at[tidx_v.at[0]], slot0, sem0).wait()
            pltpu.sync_copy(slot0, acc_sh.at[seg_v.at[pb + half - 2]], add=True)
            pltpu.make_async_copy(wt_h.at[tidx_v.at[0]], slot1, sem1).wait()
            pltpu.sync_copy(slot1, acc_sh.at[seg_v.at[pb + half - 1]], add=True)

            pltpu.sync_copy(acc_sh.at[pl.ds(slab, hp)],
                            tout_h.at[pl.ds(base + p * hp, hp)])

    return sc_kernel(midx2d, tidx2d, seg2d, zeros, w_movie_p, w_tag_p)


def _pad_call(wt, cols_per_block):
    d, v = wt.shape
    grid = (-(-v // cols_per_block),)

    def body(w_r, o_r):
        o_r[...] = jnp.concatenate(
            [w_r[...].T,
             jnp.zeros((cols_per_block, 128 - d), jnp.float32)], axis=1)

    return pl.pallas_call(
        body,
        grid=grid,
        in_specs=[pl.BlockSpec((d, cols_per_block), lambda i: (0, i))],
        out_specs=pl.BlockSpec((cols_per_block, 128), lambda i: (i, 0)),
        out_shape=jax.ShapeDtypeStruct((v, 128), jnp.float32),
    )(wt)


def _user_elem_gather(W_user, users):
    b = users.shape[0]
    d = W_user.shape[1]
    rows = jnp.broadcast_to(users.astype(jnp.int32)[:, None], (b, d))
    cols = jnp.broadcast_to(jnp.arange(d, dtype=jnp.int32)[None, :], (b, d))
    idx = jnp.stack([rows, cols], axis=-1).reshape(b * d, 2)
    dn = lax.GatherDimensionNumbers(
        offset_dims=(), collapsed_slice_dims=(0, 1), start_index_map=(0, 1))
    out = lax.gather(W_user, idx, dn, (1, 1),
                     mode=lax.GatherScatterMode.PROMISE_IN_BOUNDS)
    return out.reshape(b, d)


def _mlp_call(u, m, g, t, sc_, w1u, w1m, w1g, w1t, w1s, b1r, w2r, b2r, inv_hist):
    b = m.shape[0]
    rb = 2048
    grid = (b // rb,)

    def body(u_r, m_r, g_r, t_r, s_r, w1u_r, w1m_r, w1g_r, w1t_r, w1s_r,
             b1_r, w2_r, b2_r, o_r):
        h = jnp.dot(u_r[...], w1u_r[...], preferred_element_type=jnp.float32)
        h = h + jnp.dot(m_r[...], w1m_r[...], preferred_element_type=jnp.float32)
        h = h + jnp.dot(g_r[...], w1g_r[...], preferred_element_type=jnp.float32)
        h = h + jnp.dot(t_r[...] * inv_hist, w1t_r[...],
                        preferred_element_type=jnp.float32)
        h = h + jnp.dot(s_r[...], w1s_r[...], preferred_element_type=jnp.float32)
        h = jnp.maximum(h + b1_r[...], 0.0)
        logit = jnp.sum(h * w2_r[...], axis=1) + b2_r[0, 0]
        o_r[...] = jax.nn.sigmoid(logit) * 4.5 + 0.5

    def rowspec(d):
        return pl.BlockSpec((rb, d), lambda i: (i, 0))

    def fullspec(r, d):
        return pl.BlockSpec((r, d), lambda i: (0, 0))

    return pl.pallas_call(
        body,
        grid=grid,
        in_specs=[
            rowspec(u.shape[1]),
            rowspec(m.shape[1]), rowspec(g.shape[1]),
            rowspec(t.shape[1]), rowspec(sc_.shape[1]),
            fullspec(*w1u.shape), fullspec(*w1m.shape), fullspec(*w1g.shape),
            fullspec(*w1t.shape), fullspec(*w1s.shape),
            fullspec(*b1r.shape), fullspec(*w2r.shape), fullspec(*b2r.shape),
        ],
        out_specs=pl.BlockSpec((rb,), lambda i: (i,)),
        out_shape=jax.ShapeDtypeStruct((b,), jnp.float32),
    )(u, m, g, t, sc_, w1u, w1m, w1g, w1t, w1s, b1r, w2r, b2r)


def kernel(users, movies, genres, tags, lang, budget, popularity, runtime,
           vote_average, vote_count, revenue, W_user, W_movie, W_tag, W1, b1, W2, b2):
    b = users.shape[0]
    hist = tags.shape[1]
    d_um = W_user.shape[1]
    d_tag = W_tag.shape[1]
    ng = genres.shape[1]
    sc_rows = b // NC
    pw = b // NW

    w_movie_p = _pad_call(W_movie.T, 2048)
    w_tag_p = _pad_call(W_tag.T, 2048)

    midx2d = movies.astype(jnp.int32).reshape(-1, CH)
    tidx2d = tags.astype(jnp.int32).reshape(-1, CH)
    l_arr = lax.broadcasted_iota(jnp.int32, (b * hist // CH, CH), 0) * CH + \
        lax.broadcasted_iota(jnp.int32, (b * hist // CH, CH), 1)
    s_of_l = l_arr // hist
    seg2d = ((s_of_l % (b // NC)) // (b // NW)) * (b // NW // 2) + \
        (s_of_l % (b // NW // 2))
    zeros = jnp.zeros((b // NW // 2, 128), jnp.float32)

    movie_emb, tag_sum = _sc_gather_call(
        midx2d, tidx2d, seg2d, zeros, w_movie_p, w_tag_p, b, hist)

    user_emb = _user_elem_gather(W_user, users)

    scal = jnp.stack(
        [lang.astype(jnp.float32), budget, popularity, runtime,
         vote_average, vote_count, revenue], axis=1)

    w1u = W1[:, :d_um].T
    w1m = jnp.zeros((128, 128), jnp.float32).at[:d_um].set(W1[:, d_um:2 * d_um].T)
    w1g = W1[:, 2 * d_um:2 * d_um + ng].T
    w1t = jnp.zeros((128, 128), jnp.float32).at[:d_tag].set(
        W1[:, 2 * d_um + ng:2 * d_um + ng + d_tag].T)
    w1s = W1[:, 2 * d_um + ng + d_tag:].T
    b1r = b1.reshape(1, -1)
    b2r = b2.reshape(1, 1)

    return _mlp_call(user_emb, movie_emb, genres, tag_sum, scal,
                     w1u, w1m, w1g, w1t, w1s, b1r, W2, b2r, 1.0 / hist)


# speedup vs baseline: 7.5037x; 1.0201x over previous
import functools

import jax
import jax.numpy as jnp
from jax import lax
from jax.experimental import pallas as pl
from jax.experimental.pallas import tpu as pltpu
from jax.experimental.pallas import tpu_sc as plsc

NC = 2
NS = 16
NW = NC * NS
CH = 128


def _sc_gather_call(midx2d, tidx2d, seg2d, zeros, w_movie_p, w_tag_p, b, hist):
    pw = b // NW
    hp = pw // 2
    mch = pw // CH
    tch = pw * hist // CH
    half = tch // 2
    acc_rows = NS * hp

    mesh = plsc.VectorSubcoreMesh(
        core_axis_name="c", subcore_axis_name="s", num_cores=NC, num_subcores=NS
    )

    @functools.partial(
        pl.kernel,
        out_type=(
            jax.ShapeDtypeStruct((b, 128), jnp.float32),
            jax.ShapeDtypeStruct((b, 128), jnp.float32),
        ),
        mesh=mesh,
        scratch_types=(
            pltpu.VMEM((half + 4, CH), jnp.int32),
            pltpu.VMEM((half + 4, CH), jnp.int32),
            pltpu.VMEM((mch, CH), jnp.int32),
            pltpu.VMEM((4 * CH, 128), jnp.float32),
            pltpu.VMEM_SHARED((acc_rows, 128), jnp.float32),
            pltpu.SemaphoreType.DMA,
            pltpu.SemaphoreType.DMA,
            pltpu.SemaphoreType.DMA,
            pltpu.SemaphoreType.DMA,
            pltpu.SemaphoreType.DMA,
            pltpu.SemaphoreType.DMA,
            pltpu.SemaphoreType.DMA,
            pltpu.SemaphoreType.DMA,
        ),
    )
    def sc_kernel(midx_h, tidx_h, seg_h, zeros_h, wm_h, wt_h,
                  mout_h, tout_h,
                  tidx_v, seg_v, midx_v, ring_v, acc_sh,
                  gs0, gs1, gs2, gs3, ss0, ss1, ss2, ss3):
        c = lax.axis_index("c")
        s = lax.axis_index("s")
        wid = c * NS + s
        base = wid * pw
        slab = s * hp

        gsem = (gs0, gs1, gs2, gs3)
        ssem = (ss0, ss1, ss2, ss3)
        slots = tuple(ring_v.at[pl.ds(j * CH, CH)] for j in range(4))

        def gwait(j):
            pltpu.make_async_copy(wt_h.at[tidx_v.at[0]], slots[j], gsem[j]).wait()

        def swait(j):
            pltpu.make_async_copy(wt_h.at[tidx_v.at[0]], slots[j], ssem[j]).wait()

        pltpu.sync_copy(midx_h.at[pl.ds(wid * mch, mch)], midx_v)
        pltpu.async_copy(wm_h.at[midx_v.at[0]], slots[0], gs0)
        pltpu.async_copy(wm_h.at[midx_v.at[1]], slots[1], gs1)
        for j in range(mch):
            slot = slots[j % 2]
            sem = gsem[j % 2]
            pltpu.make_async_copy(wm_h.at[midx_v.at[0]], slot, sem).wait()
            pltpu.sync_copy(slot, mout_h.at[pl.ds(base + j * CH, CH)])
            if j + 2 < mch:
                pltpu.async_copy(wm_h.at[midx_v.at[j + 2]], slot, sem)

        for p in range(2):
            loff = 4 * p

            def gissue(cc, j, loff=loff):
                pltpu.async_copy(wt_h.at[tidx_v.at[cc + loff]], slots[j], gsem[j])

            def sissue(cc, j, loff=loff):
                pltpu.async_copy(slots[j], acc_sh.at[seg_v.at[cc + loff]],
                                 ssem[j], add=True)

            pltpu.sync_copy(
                tidx_h.at[pl.ds(wid * tch + p * (half - 4), half + 4)], tidx_v)
            pltpu.sync_copy(
                seg_h.at[pl.ds(wid * tch + p * (half - 4), half + 4)], seg_v)
            gissue(0, 0)
            gissue(1, 1)
            pltpu.sync_copy(zeros_h, acc_sh.at[pl.ds(slab, hp)])

            gwait(0); sissue(0, 0); gissue(2, 2)
            gwait(1); sissue(1, 1); gissue(3, 3)
            gwait(2); sissue(2, 2); swait(0); gissue(4, 0)
            gwait(3); sissue(3, 3); swait(1); gissue(5, 1)

            def chunk_step(i, carry):
                k = 4 * i
                gwait(0); sissue(k, 0); swait(2); gissue(k + 2, 2)
                gwait(1); sissue(k + 1, 1); swait(3); gissue(k + 3, 3)
                gwait(2); sissue(k + 2, 2); swait(0); gissue(k + 4, 0)
                gwait(3); sissue(k + 3, 3); swait(1); gissue(k + 5, 1)
                return carry

            lax.fori_loop(1, half // 4 - 1, chunk_step, 0)

            e = half - 4
            gwait(0); sissue(e, 0); swait(2); gissue(e + 2, 2)
            gwait(1); sissue(e + 1, 1); swait(3); gissue(e + 3, 3)
            gwait(2); sissue(e + 2, 2); swait(0)
            gwait(3); sissue(e + 3, 3); swait(1)
            swait(2)
            swait(3)

            pltpu.sync_copy(acc_sh.at[pl.ds(slab, hp)],
                            tout_h.at[pl.ds(base + p * hp, hp)])

    return sc_kernel(midx2d, tidx2d, seg2d, zeros, w_movie_p, w_tag_p)


def _pad_call(wt, cols_per_block):
    d, v = wt.shape
    grid = (-(-v // cols_per_block),)

    def body(w_r, o_r):
        o_r[...] = jnp.concatenate(
            [w_r[...].T,
             jnp.zeros((cols_per_block, 128 - d), jnp.float32)], axis=1)

    return pl.pallas_call(
        body,
        grid=grid,
        in_specs=[pl.BlockSpec((d, cols_per_block), lambda i: (0, i))],
        out_specs=pl.BlockSpec((cols_per_block, 128), lambda i: (i, 0)),
        out_shape=jax.ShapeDtypeStruct((v, 128), jnp.float32),
    )(wt)


def _user_elem_gather(W_user, users):
    b = users.shape[0]
    d = W_user.shape[1]
    rows = jnp.broadcast_to(users.astype(jnp.int32)[:, None], (b, d))
    cols = jnp.broadcast_to(jnp.arange(d, dtype=jnp.int32)[None, :], (b, d))
    idx = jnp.stack([rows, cols], axis=-1).reshape(b * d, 2)
    dn = lax.GatherDimensionNumbers(
        offset_dims=(), collapsed_slice_dims=(0, 1), start_index_map=(0, 1))
    out = lax.gather(W_user, idx, dn, (1, 1),
                     mode=lax.GatherScatterMode.PROMISE_IN_BOUNDS)
    return out.reshape(b, d)


def _mlp_call(u, m, g, t, sc_, w1u, w1m, w1g, w1t, w1s, b1r, w2r, b2r, inv_hist):
    b = m.shape[0]
    rb = 2048
    grid = (b // rb,)

    def body(u_r, m_r, g_r, t_r, s_r, w1u_r, w1m_r, w1g_r, w1t_r, w1s_r,
             b1_r, w2_r, b2_r, o_r):
        h = jnp.dot(u_r[...], w1u_r[...], preferred_element_type=jnp.float32)
        h = h + jnp.dot(m_r[...], w1m_r[...], preferred_element_type=jnp.float32)
        h = h + jnp.dot(g_r[...], w1g_r[...], preferred_element_type=jnp.float32)
        h = h + jnp.dot(t_r[...] * inv_hist, w1t_r[...],
                        preferred_element_type=jnp.float32)
        h = h + jnp.dot(s_r[...], w1s_r[...], preferred_element_type=jnp.float32)
        h = jnp.maximum(h + b1_r[...], 0.0)
        logit = jnp.sum(h * w2_r[...], axis=1) + b2_r[0, 0]
        o_r[...] = jax.nn.sigmoid(logit) * 4.5 + 0.5

    def rowspec(d):
        return pl.BlockSpec((rb, d), lambda i: (i, 0))

    def fullspec(r, d):
        return pl.BlockSpec((r, d), lambda i: (0, 0))

    return pl.pallas_call(
        body,
        grid=grid,
        in_specs=[
            rowspec(u.shape[1]),
            rowspec(m.shape[1]), rowspec(g.shape[1]),
            rowspec(t.shape[1]), rowspec(sc_.shape[1]),
            fullspec(*w1u.shape), fullspec(*w1m.shape), fullspec(*w1g.shape),
            fullspec(*w1t.shape), fullspec(*w1s.shape),
            fullspec(*b1r.shape), fullspec(*w2r.shape), fullspec(*b2r.shape),
        ],
        out_specs=pl.BlockSpec((rb,), lambda i: (i,)),
        out_shape=jax.ShapeDtypeStruct((b,), jnp.float32),
    )(u, m, g, t, sc_, w1u, w1m, w1g, w1t, w1s, b1r, w2r, b2r)


def kernel(users, movies, genres, tags, lang, budget, popularity, runtime,
           vote_average, vote_count, revenue, W_user, W_movie, W_tag, W1, b1, W2, b2):
    b = users.shape[0]
    hist = tags.shape[1]
    d_um = W_user.shape[1]
    d_tag = W_tag.shape[1]
    ng = genres.shape[1]
    sc_rows = b // NC
    pw = b // NW

    w_movie_p = _pad_call(W_movie.T, 2048)
    w_tag_p = _pad_call(W_tag.T, 2048)

    midx2d = movies.astype(jnp.int32).reshape(-1, CH)
    tidx2d = tags.astype(jnp.int32).reshape(-1, CH)
    l_arr = lax.broadcasted_iota(jnp.int32, (b * hist // CH, CH), 0) * CH + \
        lax.broadcasted_iota(jnp.int32, (b * hist // CH, CH), 1)
    s_of_l = l_arr // hist
    seg2d = ((s_of_l % (b // NC)) // (b // NW)) * (b // NW // 2) + \
        (s_of_l % (b // NW // 2))
    zeros = jnp.zeros((b // NW // 2, 128), jnp.float32)

    movie_emb, tag_sum = _sc_gather_call(
        midx2d, tidx2d, seg2d, zeros, w_movie_p, w_tag_p, b, hist)

    user_emb = _user_elem_gather(W_user, users)

    scal = jnp.stack(
        [lang.astype(jnp.float32), budget, popularity, runtime,
         vote_average, vote_count, revenue], axis=1)

    w1u = W1[:, :d_um].T
    w1m = jnp.zeros((128, 128), jnp.float32).at[:d_um].set(W1[:, d_um:2 * d_um].T)
    w1g = W1[:, 2 * d_um:2 * d_um + ng].T
    w1t = jnp.zeros((128, 128), jnp.float32).at[:d_tag].set(
        W1[:, 2 * d_um + ng:2 * d_um + ng + d_tag].T)
    w1s = W1[:, 2 * d_um + ng + d_tag:].T
    b1r = b1.reshape(1, -1)
    b2r = b2.reshape(1, 1)

    return _mlp_call(user_emb, movie_emb, genres, tag_sum, scal,
                     w1u, w1m, w1g, w1t, w1s, b1r, W2, b2r, 1.0 / hist)
